# baseline (speedup 1.0000x reference)
import jax
import jax.numpy as jnp
from jax.experimental import pallas as pl
from jax.experimental.pallas import tpu as pltpu

EPS = 1e-06
SUB = 64
NSUB = 2048 // SUB
NBUF = 16
CHUNK = 64

_LOG2E = 1.4426950408889634
_LN2 = 0.6931471805599453


def _clamp_kernel(mask_ref, ub_ref, x_hbm, o_hbm, bufs, sin, sout):
    m = mask_ref[...] > 0.5
    ub = ub_ref[0, 0]

    def copy_in(g, b):
        return pltpu.make_async_copy(
            x_hbm.at[pl.ds(g * SUB, SUB)], bufs[b], sin[b])

    def copy_out(g, b):
        return pltpu.make_async_copy(
            bufs[b], o_hbm.at[pl.ds(g * SUB, SUB)], sout[b])

    def compute(b):
        buf = bufs[b]

        def body(k, _):
            x = buf[pl.ds(k * CHUNK, CHUNK)]
            a = jnp.abs(x)
            e = jnp.exp2(a * (-_LOG2E))
            ls = jnp.minimum(x, 0.0) - _LN2 * jnp.log2(1.0 + e)
            buf[pl.ds(k * CHUNK, CHUNK)] = jnp.where(m, ub + ls, x)
            return 0

        jax.lax.fori_loop(0, SUB // CHUNK, body, 0, unroll=False)

    for b in range(NBUF - 1):
        copy_in(b, b).start()
    for g in range(NSUB):
        b = g % NBUF
        copy_in(g, b).wait()
        compute(b)
        copy_out(g, b).start()
        ng = g + NBUF - 1
        if ng < NSUB:
            nb = ng % NBUF
            if ng >= NBUF:
                copy_out(ng - NBUF, nb).wait()
            copy_in(ng, nb).start()
    for g in range(NSUB - NBUF, NSUB):
        copy_out(g, g % NBUF).wait()


def kernel(x, bounded_col_idx, upper_bounds):
    n_rows, n_cols = x.shape
    x3 = x.reshape(n_rows // 8, 8, n_cols)
    mask = jnp.zeros((n_cols,), jnp.float32).at[bounded_col_idx].set(1.0)
    mask3 = jnp.broadcast_to(mask, (1, 8, n_cols))
    ub2d = (jnp.asarray(upper_bounds, jnp.float32) - EPS).reshape(1, 1)
    out = pl.pallas_call(
        _clamp_kernel,
        grid=(1,),
        in_specs=[
            pl.BlockSpec((1, 8, n_cols), lambda i: (0, 0, 0)),
            pl.BlockSpec((1, 1), lambda i: (0, 0)),
            pl.BlockSpec(memory_space=pl.ANY),
        ],
        out_specs=pl.BlockSpec(memory_space=pl.ANY),
        out_shape=jax.ShapeDtypeStruct(x3.shape, x.dtype),
        scratch_shapes=(
            [pltpu.VMEM((SUB, 8, n_cols), jnp.float32) for _ in range(NBUF)],
            [pltpu.SemaphoreType.DMA for _ in range(NBUF)],
            [pltpu.SemaphoreType.DMA for _ in range(NBUF)],
        ),
    )(mask3, ub2d, x3)
    return out.reshape(n_rows, n_cols)


# speedup vs baseline: 1.0441x; 1.0441x over previous
import jax
import jax.numpy as jnp
from jax.experimental import pallas as pl
from jax.experimental.pallas import tpu as pltpu

EPS = 1e-06
SUB = 128
NSUB = 2048 // SUB
NBUF = 8
CHUNK = 64

_LOG2E = 1.4426950408889634
_LN2 = 0.6931471805599453


def _clamp_kernel(mask_ref, ub_ref, x_hbm, o_hbm, bufs, sin, sout):
    m = mask_ref[...] > 0.5
    ub = ub_ref[0, 0]

    def copy_in(g, b):
        return pltpu.make_async_copy(
            x_hbm.at[pl.ds(g * SUB, SUB)], bufs[b], sin[b])

    def copy_out(g, b):
        return pltpu.make_async_copy(
            bufs[b], o_hbm.at[pl.ds(g * SUB, SUB)], sout[b])

    def compute(b):
        buf = bufs[b]

        def body(k, _):
            x = buf[pl.ds(k * CHUNK, CHUNK)]
            a = jnp.abs(x)
            e = jnp.exp2(a * (-_LOG2E))
            ls = jnp.minimum(x, 0.0) - _LN2 * jnp.log2(1.0 + e)
            buf[pl.ds(k * CHUNK, CHUNK)] = jnp.where(m, ub + ls, x)
            return 0

        jax.lax.fori_loop(0, SUB // CHUNK, body, 0, unroll=False)

    for b in range(NBUF - 1):
        copy_in(b, b).start()
    for g in range(NSUB):
        b = g % NBUF
        copy_in(g, b).wait()
        compute(b)
        copy_out(g, b).start()
        ng = g + NBUF - 1
        if ng < NSUB:
            nb = ng % NBUF
            if ng >= NBUF:
                copy_out(ng - NBUF, nb).wait()
            copy_in(ng, nb).start()
    for g in range(NSUB - NBUF, NSUB):
        copy_out(g, g % NBUF).wait()


def kernel(x, bounded_col_idx, upper_bounds):
    n_rows, n_cols = x.shape
    x3 = x.reshape(n_rows // 8, 8, n_cols)
    mask = jnp.zeros((n_cols,), jnp.float32).at[bounded_col_idx].set(1.0)
    mask3 = jnp.broadcast_to(mask, (1, 8, n_cols))
    ub2d = (jnp.asarray(upper_bounds, jnp.float32) - EPS).reshape(1, 1)
    out = pl.pallas_call(
        _clamp_kernel,
        grid=(1,),
        in_specs=[
            pl.BlockSpec((1, 8, n_cols), lambda i: (0, 0, 0)),
            pl.BlockSpec((1, 1), lambda i: (0, 0)),
            pl.BlockSpec(memory_space=pl.ANY),
        ],
        out_specs=pl.BlockSpec(memory_space=pl.ANY),
        out_shape=jax.ShapeDtypeStruct(x3.shape, x.dtype),
        scratch_shapes=(
            [pltpu.VMEM((SUB, 8, n_cols), jnp.float32) for _ in range(NBUF)],
            [pltpu.SemaphoreType.DMA for _ in range(NBUF)],
            [pltpu.SemaphoreType.DMA for _ in range(NBUF)],
        ),
    )(mask3, ub2d, x3)
    return out.reshape(n_rows, n_cols)
